# Initial kernel scaffold; baseline (speedup 1.0000x reference)
import jax
import jax.numpy as jnp
from jax.experimental import pallas as pl


def kernel(output_map_0, output_map_1, gt_density, process):
    raise NotImplementedError("write your pallas kernel here")


# speedup vs baseline: 21.6805x; 21.6805x over previous
import functools

import jax
import jax.numpy as jnp
from jax.experimental import pallas as pl
from jax.experimental.pallas import tpu as pltpu

_POOL = 8


def _pool_kernel(gt_ref, dmap_ref, *, rows_in, cols_in, rows_out, cols_out):
    x = gt_ref[0, 0]
    io = jax.lax.broadcasted_iota
    ph = (io(jnp.int32, (rows_out, rows_in), 1) // _POOL
          == io(jnp.int32, (rows_out, rows_in), 0)).astype(jnp.float32)
    xh = jnp.dot(ph, x, preferred_element_type=jnp.float32)
    pw = (io(jnp.int32, (cols_in, cols_out), 0) // _POOL
          == io(jnp.int32, (cols_in, cols_out), 1)).astype(jnp.float32)
    dmap_ref[0] = jnp.dot(xh, pw, preferred_element_type=jnp.float32)


def _loss_kernel(om0_ref, om1_ref, dmap_ref, w_ref, out_ref, bits_ref, *,
                 num, rows):
    om0 = om0_ref[...]
    dmap = dmap_ref[...]
    err = jnp.abs(dmap - om0)
    bits_ref[...] = jax.lax.bitcast_convert_type(err, jnp.int32)

    def body(i, res):
        cand = res | (jnp.int32(1) << (jnp.int32(30) - i))
        cnt = jnp.sum((bits_ref[...] >= cand).astype(jnp.int32),
                      axis=1, keepdims=True)
        return jnp.where(cnt >= num, cand, res)

    thr = jax.lax.fori_loop(0, 31, body, jnp.zeros((rows, 1), jnp.int32))
    sel = bits_ref[...] >= thr
    w = w_ref[0, 0]
    comb = w * om1_ref[...] + (1.0 - w) * dmap
    d = om0 - jnp.where(sel, comb, dmap)
    out_ref[...] = jnp.sum(d * d, keepdims=True)


def kernel(output_map_0, output_map_1, gt_density, process):
    b, c, h, w = output_map_0.shape
    B, C, H, W = gt_density.shape
    n = c * h * w
    num = int(h * w * 0.1)

    rows_in = 256
    rows_out = rows_in // _POOL
    n_chunks = H // rows_in
    dmap = pl.pallas_call(
        functools.partial(_pool_kernel, rows_in=rows_in, cols_in=W,
                          rows_out=rows_out, cols_out=w),
        grid=(B, n_chunks),
        in_specs=[pl.BlockSpec((1, 1, rows_in, W),
                               lambda bi, hi: (bi, 0, hi, 0))],
        out_specs=pl.BlockSpec((1, rows_out, w), lambda bi, hi: (bi, hi, 0)),
        out_shape=jax.ShapeDtypeStruct((B, h, w), jnp.float32),
    )(gt_density)

    dmap2 = dmap.reshape(B, n)
    om0 = output_map_0.reshape(b, n)
    om1 = output_map_1.reshape(b, n)
    wmat = jnp.asarray(process, jnp.float32).reshape(1, 1)

    full = lambda s: pl.BlockSpec(s, lambda i: tuple(0 for _ in s))
    loss = pl.pallas_call(
        functools.partial(_loss_kernel, num=num, rows=b),
        grid=(1,),
        in_specs=[full((b, n)), full((b, n)), full((b, n)), full((1, 1))],
        out_specs=full((1, 1)),
        out_shape=jax.ShapeDtypeStruct((1, 1), jnp.float32),
        scratch_shapes=[pltpu.VMEM((b, n), jnp.int32)],
    )(om0, om1, dmap2, wmat)
    return loss[0, 0]


# speedup vs baseline: 27.3660x; 1.2622x over previous
import functools

import jax
import jax.numpy as jnp
from jax.experimental import pallas as pl
from jax.experimental.pallas import tpu as pltpu

_POOL = 8


def _pool_kernel(gt_ref, dmap_ref, *, rows_in, cols_in, rows_out, cols_out):
    x = gt_ref[0, 0]
    io = jax.lax.broadcasted_iota
    ph = (io(jnp.int32, (rows_out, rows_in), 1) // _POOL
          == io(jnp.int32, (rows_out, rows_in), 0)).astype(jnp.float32)
    xh = jnp.dot(ph, x, preferred_element_type=jnp.float32)
    pw = (io(jnp.int32, (cols_in, cols_out), 0) // _POOL
          == io(jnp.int32, (cols_in, cols_out), 1)).astype(jnp.float32)
    dmap_ref[0] = jnp.dot(xh, pw, preferred_element_type=jnp.float32)


def _loss_kernel(om0_ref, om1_ref, dmap_ref, w_ref, out_ref, bits_ref, *,
                 num, rows):
    om0 = om0_ref[...]
    dmap = dmap_ref[...]
    err = jnp.abs(dmap - om0)
    bits_ref[...] = jax.lax.bitcast_convert_type(err, jnp.int32)

    def body(i, res):
        cand = res | (jnp.int32(1) << (jnp.int32(30) - i))
        cnt = jnp.sum((bits_ref[...] >= cand).astype(jnp.int32),
                      axis=1, keepdims=True)
        return jnp.where(cnt >= num, cand, res)

    thr = jax.lax.fori_loop(0, 31, body, jnp.zeros((rows, 1), jnp.int32))
    sel = bits_ref[...] >= thr
    w = w_ref[0, 0]
    comb = w * om1_ref[...] + (1.0 - w) * dmap
    d = om0 - jnp.where(sel, comb, dmap)
    out_ref[...] = jnp.sum(d * d, keepdims=True)


def kernel(output_map_0, output_map_1, gt_density, process):
    b, c, h, w = output_map_0.shape
    B, C, H, W = gt_density.shape
    n = c * h * w
    num = int(h * w * 0.1)

    rows_in = 512
    rows_out = rows_in // _POOL
    n_chunks = H // rows_in
    dmap = pl.pallas_call(
        functools.partial(_pool_kernel, rows_in=rows_in, cols_in=W,
                          rows_out=rows_out, cols_out=w),
        grid=(B, n_chunks),
        in_specs=[pl.BlockSpec((1, 1, rows_in, W),
                               lambda bi, hi: (bi, 0, hi, 0))],
        out_specs=pl.BlockSpec((1, rows_out, w), lambda bi, hi: (bi, hi, 0)),
        out_shape=jax.ShapeDtypeStruct((B, h, w), jnp.float32),
    )(gt_density)

    dmap2 = dmap.reshape(B, n)
    om0 = output_map_0.reshape(b, n)
    om1 = output_map_1.reshape(b, n)
    wmat = jnp.asarray(process, jnp.float32).reshape(1, 1)

    full = lambda s: pl.BlockSpec(s, lambda i: tuple(0 for _ in s))
    loss = pl.pallas_call(
        functools.partial(_loss_kernel, num=num, rows=b),
        grid=(1,),
        in_specs=[full((b, n)), full((b, n)), full((b, n)), full((1, 1))],
        out_specs=full((1, 1)),
        out_shape=jax.ShapeDtypeStruct((1, 1), jnp.float32),
        scratch_shapes=[pltpu.VMEM((b, n), jnp.int32)],
    )(om0, om1, dmap2, wmat)
    return loss[0, 0]


# speedup vs baseline: 30.1009x; 1.0999x over previous
import functools

import jax
import jax.numpy as jnp
from jax.experimental import pallas as pl
from jax.experimental.pallas import tpu as pltpu

_POOL = 8


def _pool_kernel(gt_ref, dmap_ref, *, rows_in, cols_in, rows_out, cols_out):
    x = gt_ref[0, 0]
    io = jax.lax.broadcasted_iota
    ph = (io(jnp.int32, (rows_out, rows_in), 1) // _POOL
          == io(jnp.int32, (rows_out, rows_in), 0)).astype(jnp.float32)
    xh = jnp.dot(ph, x, preferred_element_type=jnp.float32)
    pw = (io(jnp.int32, (cols_in, cols_out), 0) // _POOL
          == io(jnp.int32, (cols_in, cols_out), 1)).astype(jnp.float32)
    dmap_ref[0] = jnp.dot(xh, pw, preferred_element_type=jnp.float32)


def _loss_kernel(om0_ref, om1_ref, dmap_ref, w_ref, out_ref, bits_ref, *,
                 num, rows):
    om0 = om0_ref[...]
    dmap = dmap_ref[...]
    err = jnp.abs(dmap - om0)
    bits_ref[...] = jax.lax.bitcast_convert_type(err, jnp.int32)

    def body(i, res):
        cand = res | (jnp.int32(1) << (jnp.int32(30) - i))
        cnt = jnp.sum((bits_ref[...] >= cand).astype(jnp.int32),
                      axis=1, keepdims=True)
        return jnp.where(cnt >= num, cand, res)

    thr = jax.lax.fori_loop(0, 31, body, jnp.zeros((rows, 1), jnp.int32))
    sel = bits_ref[...] >= thr
    w = w_ref[0, 0]
    comb = w * om1_ref[...] + (1.0 - w) * dmap
    d = om0 - jnp.where(sel, comb, dmap)
    out_ref[...] = jnp.sum(d * d, keepdims=True)


def kernel(output_map_0, output_map_1, gt_density, process):
    b, c, h, w = output_map_0.shape
    B, C, H, W = gt_density.shape
    n = c * h * w
    num = int(h * w * 0.1)

    rows_in = 1024
    rows_out = rows_in // _POOL
    n_chunks = H // rows_in
    dmap = pl.pallas_call(
        functools.partial(_pool_kernel, rows_in=rows_in, cols_in=W,
                          rows_out=rows_out, cols_out=w),
        grid=(B, n_chunks),
        in_specs=[pl.BlockSpec((1, 1, rows_in, W),
                               lambda bi, hi: (bi, 0, hi, 0))],
        out_specs=pl.BlockSpec((1, rows_out, w), lambda bi, hi: (bi, hi, 0)),
        out_shape=jax.ShapeDtypeStruct((B, h, w), jnp.float32),
    )(gt_density)

    dmap2 = dmap.reshape(B, n)
    om0 = output_map_0.reshape(b, n)
    om1 = output_map_1.reshape(b, n)
    wmat = jnp.asarray(process, jnp.float32).reshape(1, 1)

    full = lambda s: pl.BlockSpec(s, lambda i: tuple(0 for _ in s))
    loss = pl.pallas_call(
        functools.partial(_loss_kernel, num=num, rows=b),
        grid=(1,),
        in_specs=[full((b, n)), full((b, n)), full((b, n)), full((1, 1))],
        out_specs=full((1, 1)),
        out_shape=jax.ShapeDtypeStruct((1, 1), jnp.float32),
        scratch_shapes=[pltpu.VMEM((b, n), jnp.int32)],
    )(om0, om1, dmap2, wmat)
    return loss[0, 0]
